# baseline (device time: 7115 ns/iter reference)
import jax
import jax.numpy as jnp
from jax import lax
from jax.experimental import pallas as pl
from jax.experimental.pallas import tpu as pltpu

N_DEV = 4


def kernel(x):
    m_per, n = x.shape
    inv_total = 1.0 / float(N_DEV * m_per)

    half = m_per // 2

    def body(x_ref, out_ref, xv_ref, mine_ref, comm_ref,
             load_sems, send_sems, recv_sems):
        my = lax.axis_index("i")

        barrier_sem = pltpu.get_barrier_semaphore()
        for d in range(1, N_DEV):
            peer = lax.rem(my + d, N_DEV)
            pl.semaphore_signal(
                barrier_sem, inc=1,
                device_id=(peer,), device_id_type=pl.DeviceIdType.MESH,
            )

        c0 = pltpu.make_async_copy(
            x_ref.at[pl.ds(0, half)], xv_ref.at[0], load_sems.at[0])
        c1 = pltpu.make_async_copy(
            x_ref.at[pl.ds(half, half)], xv_ref.at[1], load_sems.at[1])
        c0.start()
        c1.start()
        c0.wait()
        s0 = jnp.sum(xv_ref[0, :, :], axis=0, keepdims=True)
        c1.wait()
        mine_ref[:, :] = s0 + jnp.sum(xv_ref[1, :, :], axis=0, keepdims=True)

        pl.semaphore_wait(barrier_sem, N_DEV - 1)

        rdmas = []
        for d in range(1, N_DEV):
            peer = lax.rem(my + d, N_DEV)
            rdma = pltpu.make_async_remote_copy(
                src_ref=mine_ref,
                dst_ref=comm_ref.at[d - 1],
                send_sem=send_sems.at[d - 1],
                recv_sem=recv_sems.at[d - 1],
                device_id=(peer,),
                device_id_type=pl.DeviceIdType.MESH,
            )
            rdma.start()
            rdmas.append(rdma)

        for rdma in rdmas:
            rdma.wait_recv()
        for rdma in rdmas:
            rdma.wait_send()

        total = (
            mine_ref[:, :]
            + comm_ref[0, :, :]
            + comm_ref[1, :, :]
            + comm_ref[2, :, :]
        )
        out_ref[:, :] = total * inv_total

    return pl.pallas_call(
        body,
        out_shape=jax.ShapeDtypeStruct((1, n), jnp.float32),
        in_specs=[pl.BlockSpec(memory_space=pl.ANY)],
        out_specs=pl.BlockSpec(memory_space=pltpu.VMEM),
        scratch_shapes=[
            pltpu.VMEM((2, half, n), jnp.float32),
            pltpu.VMEM((1, n), jnp.float32),
            pltpu.VMEM((N_DEV - 1, 1, n), jnp.float32),
            pltpu.SemaphoreType.DMA((2,)),
            pltpu.SemaphoreType.DMA((N_DEV - 1,)),
            pltpu.SemaphoreType.DMA((N_DEV - 1,)),
        ],
        compiler_params=pltpu.CompilerParams(collective_id=0),
    )(x)
